# Initial kernel scaffold; baseline (speedup 1.0000x reference)
import jax
import jax.numpy as jnp
from jax.experimental import pallas as pl


def kernel(x, edge_index, edge_attr, W_x, b_x, W_e, b_e, W_m, b_m, beta):
    raise NotImplementedError("write your pallas kernel here")


# speedup vs baseline: 3.2810x; 3.2810x over previous
import jax
import jax.numpy as jnp
from jax import lax
from jax.experimental import pallas as pl
from jax.experimental.pallas import tpu as pltpu
from jax.experimental.pallas import tpu_sc as plsc

N = 10000
E = 320000
D_FEAT = 128
D_EDGE = 16
OUT = 128

NC = 2
NS = 16
NW = NC * NS
LANES = 16

CH = 128
NCHUNKS = E // CH
FULL = NCHUNKS // NW
EXTRA = NCHUNKS - FULL * NW
RPT = 624
REM = N - NS * RPT
ZROWS = 104


def _h2_body(x_ref, wx_ref, bx_ref, wm1_ref, o_ref):
    wxm = jnp.dot(wx_ref[...], wm1_ref[...], preferred_element_type=jnp.float32)
    bxm = jnp.dot(bx_ref[...], wm1_ref[...], preferred_element_type=jnp.float32)
    o_ref[...] = jnp.dot(x_ref[...], wxm, preferred_element_type=jnp.float32) + bxm


def _ec_body(ea_ref, we_ref, be_ref, wm2_ref, bm_ref, o_ref):
    we2 = jnp.dot(we_ref[...], wm2_ref[...], preferred_element_type=jnp.float32)
    ball = jnp.dot(be_ref[...], wm2_ref[...], preferred_element_type=jnp.float32) + bm_ref[...]
    o_ref[...] = jnp.dot(ea_ref[...], we2, preferred_element_type=jnp.float32) + ball


def _fin_body(p0_ref, p1_ref, s_ref, o_ref):
    a = p0_ref[...] + p1_ref[...]
    o_ref[...] = s_ref[0, 0] / (1.0 + jnp.exp(-a))


def _sc_body(h2_hbm, src_hbm, dst_hbm, ec_hbm, out_hbm,
             src_v, dst_v, rows_v, ec_v, acc_sh, sem):
    cid = lax.axis_index("c")
    sid = lax.axis_index("s")
    w = sid * NC + cid

    zero = jnp.zeros((LANES,), jnp.float32)

    def _z(r, carry):
        for c in range(OUT // LANES):
            ec_v[r, pl.ds(c * LANES, LANES)] = zero
        return carry
    lax.fori_loop(0, CH, _z, 0)
    for k in range(RPT // ZROWS):
        pltpu.sync_copy(ec_v.at[pl.ds(0, ZROWS)],
                        acc_sh.at[pl.ds(sid * RPT + k * ZROWS, ZROWS)])

    @pl.when(sid == NS - 1)
    def _zero_rem():
        pltpu.sync_copy(ec_v.at[pl.ds(0, REM)], acc_sh.at[pl.ds(NS * RPT, REM)])
    plsc.subcore_barrier()

    nch = FULL + jnp.where(w < EXTRA, 1, 0)

    def _chunk(i, carry):
        base = (w + i * NW) * CH
        pltpu.sync_copy(src_hbm.at[pl.ds(base, CH)], src_v)
        pltpu.sync_copy(dst_hbm.at[pl.ds(base, CH)], dst_v)
        gather = pltpu.async_copy(h2_hbm.at[src_v], rows_v, sem)
        pltpu.sync_copy(ec_hbm.at[pl.ds(base, CH)], ec_v)
        gather.wait()

        def _row(r, c2):
            for c in range(OUT // LANES):
                sl = pl.ds(c * LANES, LANES)
                v = rows_v[r, sl] + ec_v[r, sl]
                rows_v[r, sl] = jnp.maximum(v, 0.01 * v)
            return c2
        lax.fori_loop(0, CH, _row, 0)

        pltpu.sync_copy(rows_v, acc_sh.at[dst_v], add=True)
        return carry
    lax.fori_loop(0, nch, _chunk, 0)

    plsc.subcore_barrier()
    pltpu.sync_copy(acc_sh.at[pl.ds(sid * RPT, RPT)],
                    out_hbm.at[pl.ds(cid * N + sid * RPT, RPT)])

    @pl.when(sid == NS - 1)
    def _write_rem():
        pltpu.sync_copy(acc_sh.at[pl.ds(NS * RPT, REM)],
                        out_hbm.at[pl.ds(cid * N + NS * RPT, REM)])


def kernel(x, edge_index, edge_attr, W_x, b_x, W_e, b_e, W_m, b_m, beta):
    W_m1 = W_m[:D_FEAT]
    W_m2 = W_m[D_FEAT:]
    src = edge_index[0]
    dst = edge_index[1]

    h2 = pl.pallas_call(
        _h2_body,
        out_shape=jax.ShapeDtypeStruct((N, OUT), jnp.float32),
    )(x, W_x, b_x.reshape(1, OUT), W_m1)

    EB = 4000
    ec = pl.pallas_call(
        _ec_body,
        grid=(E // EB,),
        in_specs=[
            pl.BlockSpec((EB, D_EDGE), lambda i: (i, 0)),
            pl.BlockSpec((D_EDGE, OUT), lambda i: (0, 0)),
            pl.BlockSpec((1, OUT), lambda i: (0, 0)),
            pl.BlockSpec((D_FEAT, OUT), lambda i: (0, 0)),
            pl.BlockSpec((1, OUT), lambda i: (0, 0)),
        ],
        out_specs=pl.BlockSpec((EB, OUT), lambda i: (i, 0)),
        out_shape=jax.ShapeDtypeStruct((E, OUT), jnp.float32),
    )(edge_attr, W_e, b_e.reshape(1, OUT), W_m2, b_m.reshape(1, OUT))

    partial = pl.kernel(
        _sc_body,
        out_type=jax.ShapeDtypeStruct((NC * N, OUT), jnp.float32),
        mesh=plsc.VectorSubcoreMesh(core_axis_name="c", subcore_axis_name="s",
                                    num_cores=NC, num_subcores=NS),
        scratch_types=[
            pltpu.VMEM((CH,), jnp.int32),
            pltpu.VMEM((CH,), jnp.int32),
            pltpu.VMEM((CH, OUT), jnp.float32),
            pltpu.VMEM((CH, OUT), jnp.float32),
            pltpu.VMEM_SHARED((N, OUT), jnp.float32),
            pltpu.SemaphoreType.DMA,
        ],
    )(h2, src, dst, ec)

    scale = jnp.maximum(beta, 0.0).astype(jnp.float32).reshape(1, 1)
    RB = 1000
    out = pl.pallas_call(
        _fin_body,
        grid=(N // RB,),
        in_specs=[
            pl.BlockSpec((RB, OUT), lambda i: (i, 0)),
            pl.BlockSpec((RB, OUT), lambda i: (i + N // RB, 0)),
            pl.BlockSpec(memory_space=pltpu.SMEM),
        ],
        out_specs=pl.BlockSpec((RB, OUT), lambda i: (i, 0)),
        out_shape=jax.ShapeDtypeStruct((N, OUT), jnp.float32),
    )(partial, partial, scale)
    return out
